# baseline (speedup 1.0000x reference)
import functools

import jax
import jax.numpy as jnp
from jax import lax
from jax.experimental import pallas as pl
from jax.experimental.pallas import tpu as pltpu
from jax.experimental.pallas import tpu_sc as plsc

EMD_SIZE = 128
INPUT_CHANNEL = 1000
BATCH = 16384

_info = plsc.get_sparse_core_info()
_NC, _NS = _info.num_cores, _info.num_subcores
_NW = _NC * _NS
_B_PER_W = BATCH // _NW


_CH = 128
_NCHUNK = _B_PER_W // _CH


def _gather_kernel(table_hbm, idx_hbm, out_hbm, idx_v, rows_v,
                   gsem0, gsem1, wsem0, wsem1):
    wid = lax.axis_index("s") * _NC + lax.axis_index("c")
    base = wid * _B_PER_W
    pltpu.sync_copy(idx_hbm.at[pl.ds(base, _B_PER_W)], idx_v)
    gsem = (gsem0, gsem1)
    wsem = (wsem0, wsem1)
    gd = [None, None]
    wd = [None, None]
    for c in range(_NCHUNK):
        b = c % 2
        if wd[b] is not None:
            wd[b].wait()
        gd[b] = pltpu.async_copy(
            table_hbm.at[idx_v.at[pl.ds(c * _CH, _CH)]],
            rows_v.at[b], gsem[b])
        if c >= 1:
            pb = (c - 1) % 2
            gd[pb].wait()
            wd[pb] = pltpu.async_copy(
                rows_v.at[pb],
                out_hbm.at[pl.ds(base + (c - 1) * _CH, _CH)], wsem[pb])
    lb = (_NCHUNK - 1) % 2
    gd[lb].wait()
    wd[lb] = pltpu.async_copy(
        rows_v.at[lb],
        out_hbm.at[pl.ds(base + (_NCHUNK - 1) * _CH, _CH)], wsem[lb])
    wd[0].wait()
    wd[1].wait()


_mesh = plsc.VectorSubcoreMesh(core_axis_name="c", subcore_axis_name="s")

_gather = pl.kernel(
    _gather_kernel,
    mesh=_mesh,
    out_type=jax.ShapeDtypeStruct((BATCH, EMD_SIZE), jnp.float32),
    scratch_types=[
        pltpu.VMEM((_B_PER_W,), jnp.int32),
        pltpu.VMEM((2, _CH, EMD_SIZE), jnp.float32),
        pltpu.SemaphoreType.DMA,
        pltpu.SemaphoreType.DMA,
        pltpu.SemaphoreType.DMA,
        pltpu.SemaphoreType.DMA,
    ],
)


def kernel(lang, emb_weight):
    idx = lang.reshape(BATCH).astype(jnp.int32)
    out = _gather(emb_weight, idx)
    return out.reshape(BATCH, 1, EMD_SIZE)


# speedup vs baseline: 1.0296x; 1.0296x over previous
import functools

import jax
import jax.numpy as jnp
from jax import lax
from jax.experimental import pallas as pl
from jax.experimental.pallas import tpu as pltpu
from jax.experimental.pallas import tpu_sc as plsc

EMD_SIZE = 128
INPUT_CHANNEL = 1000
BATCH = 16384

_info = plsc.get_sparse_core_info()
_NC, _NS = _info.num_cores, _info.num_subcores
_NW = _NC * _NS
_B_PER_W = BATCH // _NW


_CH = _B_PER_W // 2


def _gather_kernel(table_hbm, idx_hbm, out_hbm, idx_v, rows_v,
                   gsem0, gsem1, wsem0, wsem1):
    wid = lax.axis_index("s") * _NC + lax.axis_index("c")
    base = wid * _B_PER_W
    pltpu.sync_copy(idx_hbm.at[pl.ds(base, _B_PER_W)], idx_v)
    g0 = pltpu.async_copy(table_hbm.at[idx_v.at[pl.ds(0, _CH)]],
                          rows_v.at[pl.ds(0, _CH)], gsem0)
    g1 = pltpu.async_copy(table_hbm.at[idx_v.at[pl.ds(_CH, _CH)]],
                          rows_v.at[pl.ds(_CH, _CH)], gsem1)
    g0.wait()
    w0 = pltpu.async_copy(rows_v.at[pl.ds(0, _CH)],
                          out_hbm.at[pl.ds(base, _CH)], wsem0)
    g1.wait()
    w1 = pltpu.async_copy(rows_v.at[pl.ds(_CH, _CH)],
                          out_hbm.at[pl.ds(base + _CH, _CH)], wsem1)
    w0.wait()
    w1.wait()


_mesh = plsc.VectorSubcoreMesh(core_axis_name="c", subcore_axis_name="s")

_gather = pl.kernel(
    _gather_kernel,
    mesh=_mesh,
    out_type=jax.ShapeDtypeStruct((BATCH, EMD_SIZE), jnp.float32),
    scratch_types=[
        pltpu.VMEM((_B_PER_W,), jnp.int32),
        pltpu.VMEM((_B_PER_W, EMD_SIZE), jnp.float32),
        pltpu.SemaphoreType.DMA,
        pltpu.SemaphoreType.DMA,
        pltpu.SemaphoreType.DMA,
        pltpu.SemaphoreType.DMA,
    ],
)


def kernel(lang, emb_weight):
    idx = lang.reshape(BATCH).astype(jnp.int32)
    out = _gather(emb_weight, idx)
    return out.reshape(BATCH, 1, EMD_SIZE)


# speedup vs baseline: 1.1056x; 1.0738x over previous
import functools

import jax
import jax.numpy as jnp
from jax import lax
from jax.experimental import pallas as pl
from jax.experimental.pallas import tpu as pltpu
from jax.experimental.pallas import tpu_sc as plsc

EMD_SIZE = 128
INPUT_CHANNEL = 1000
BATCH = 16384

_info = plsc.get_sparse_core_info()
_NC, _NS = _info.num_cores, _info.num_subcores
_NW = _NC * _NS
_B_PER_W = BATCH // _NW


def _gather_kernel(table_hbm, idx_hbm, out_hbm, tbl_s, idx_v, rows_v, gsem):
    sid = lax.axis_index("s")
    wid = sid * _NC + lax.axis_index("c")
    base = wid * _B_PER_W
    pltpu.sync_copy(idx_hbm.at[pl.ds(base, _B_PER_W)], idx_v)
    @pl.when(sid == 0)
    def _():
        pltpu.sync_copy(table_hbm, tbl_s)

    plsc.subcore_barrier()
    pltpu.async_copy(tbl_s.at[idx_v], rows_v, gsem).wait()
    pltpu.sync_copy(rows_v, out_hbm.at[pl.ds(base, _B_PER_W)])


_mesh = plsc.VectorSubcoreMesh(core_axis_name="c", subcore_axis_name="s")

_gather = pl.kernel(
    _gather_kernel,
    mesh=_mesh,
    out_type=jax.ShapeDtypeStruct((BATCH, EMD_SIZE), jnp.float32),
    scratch_types=[
        pltpu.VMEM_SHARED((INPUT_CHANNEL, EMD_SIZE), jnp.float32),
        pltpu.VMEM((_B_PER_W,), jnp.int32),
        pltpu.VMEM((_B_PER_W, EMD_SIZE), jnp.float32),
        pltpu.SemaphoreType.DMA,
    ],
)


def kernel(lang, emb_weight):
    idx = lang.reshape(BATCH).astype(jnp.int32)
    out = _gather(emb_weight, idx)
    return out.reshape(BATCH, 1, EMD_SIZE)
